# baseline (speedup 1.0000x reference)
import functools

import jax
import jax.numpy as jnp
from jax import lax
from jax.experimental import pallas as pl
from jax.experimental.pallas import tpu as pltpu
from jax.experimental.pallas import tpu_sc as plsc

_N_NODES = 10000
_N_EDGES = 320000
_D = 128
_NUM_REL = 1000

_NW = 32
_EPW = _N_EDGES // _NW
_E = 128
_NB = 3
_CHUNKS = _EPW // _E
_TAIL = _EPW - _CHUNKS * _E
_DP = _D // 2


def _precompute(z, phase_rel):
    def zn_body(z_ref, o_ref):
        x = z_ref[...]
        n = jnp.sqrt(jnp.sum(x * x, axis=1, keepdims=True))
        o_ref[...] = x / jnp.maximum(n, 1e-12)

    zn = pl.pallas_call(
        zn_body,
        out_shape=jax.ShapeDtypeStruct((_N_NODES, _D), jnp.float32),
        grid=(10,),
        in_specs=[pl.BlockSpec((_N_NODES // 10, _D), lambda i: (i, 0))],
        out_specs=pl.BlockSpec((_N_NODES // 10, _D), lambda i: (i, 0)),
    )(z)

    def cos_body(p_ref, o_ref):
        o_ref[...] = jnp.cos(p_ref[...])

    cosr = pl.pallas_call(
        cos_body,
        out_shape=jax.ShapeDtypeStruct((_NUM_REL, _D), jnp.float32),
    )(phase_rel)
    return zn, cosr


_mesh = plsc.VectorSubcoreMesh(core_axis_name="c", subcore_axis_name="s")


@functools.partial(
    pl.kernel,
    mesh=_mesh,
    compiler_params=pltpu.CompilerParams(needs_layout_passes=False,
                                         use_tc_tiling_on_sc=False),
    out_type=jax.ShapeDtypeStruct((_N_EDGES,), jnp.float32),
    scratch_types=[
        pltpu.VMEM((_EPW,), jnp.int32),
        pltpu.VMEM((_EPW,), jnp.int32),
        pltpu.VMEM((_EPW,), jnp.int32),
        pltpu.VMEM((_NB, _E, _DP), jnp.int32),
        pltpu.VMEM((_NB, _E, _DP), jnp.int32),
        pltpu.VMEM((_NB, _E, _DP), jnp.int32),
        pltpu.VMEM((_EPW,), jnp.float32),
        pltpu.SemaphoreType.DMA,
        pltpu.SemaphoreType.DMA,
        pltpu.SemaphoreType.DMA,
    ],
)
def _score_sc(zn_hbm, cos_hbm, src_hbm, dst_hbm, typ_hbm, out_hbm,
              src_i, dst_i, typ_i, src_r, dst_r, rel_r, out_v,
              sem0, sem1, sem2):
    wid = lax.axis_index("s") * 2 + lax.axis_index("c")
    base = pl.multiple_of(wid * _EPW, _EPW)
    pltpu.sync_copy(src_hbm.at[pl.ds(base, _EPW)], src_i)
    pltpu.sync_copy(dst_hbm.at[pl.ds(base, _EPW)], dst_i)
    pltpu.sync_copy(typ_hbm.at[pl.ds(base, _EPW)], typ_i)
    sems = (sem0, sem1, sem2)
    lane = lax.iota(jnp.int32, 16)

    def fire(c, b):
        off = pl.multiple_of(c * _E, _E)
        pltpu.async_copy(zn_hbm.at[src_i.at[pl.ds(off, _E)]], src_r.at[b], sems[b])
        pltpu.async_copy(zn_hbm.at[dst_i.at[pl.ds(off, _E)]], dst_r.at[b], sems[b])
        pltpu.async_copy(cos_hbm.at[typ_i.at[pl.ds(off, _E)]], rel_r.at[b], sems[b])

    def drain(b):
        pltpu.make_async_copy(zn_hbm.at[pl.ds(0, _E)], src_r.at[b], sems[b]).wait()
        pltpu.make_async_copy(zn_hbm.at[pl.ds(0, _E)], dst_r.at[b], sems[b]).wait()
        pltpu.make_async_copy(cos_hbm.at[pl.ds(0, _E)], rel_r.at[b], sems[b]).wait()

    def compute(c, b):
        src_f = src_r.at[b]
        dst_f = dst_r.at[b]
        rel_f = rel_r.at[b]

        def unpk(x):
            return plsc.unpack(plsc.bitcast(x, jnp.bfloat16),
                               format=plsc.PackFormat.INTERLEAVED)

        def group_body(g, carry):
            rows = g * 16 + lane

            def t_body(t, accs):
                a0, a1, col = accs
                sp = plsc.load_gather(src_f, [rows, col])
                up = plsc.load_gather(dst_f, [rows, col])
                rp = plsc.load_gather(rel_f, [rows, col])
                s0, s1 = unpk(sp)
                u0, u1 = unpk(up)
                r0, r1 = unpk(rp)
                nxt = lax.bitwise_and(col + 1, _DP - 1)
                return (a0 + s0 * u0 * r0, a1 + s1 * u1 * r1, nxt)

            z16 = jnp.zeros((16,), jnp.float32)
            a0, a1, _ = lax.fori_loop(0, _DP, t_body, (z16, z16, lane),
                                      unroll=4)
            out_v[pl.ds(c * _E + g * 16, 16)] = a0 + a1
            return carry

        lax.fori_loop(0, _E // 16, group_body, 0)

    for b in range(_NB - 1):
        fire(b, b)

    def ring_body(t, carry):
        i = t * _NB
        for b in range(_NB):
            c = i + b

            @pl.when(c + _NB - 1 < _CHUNKS)
            def _():
                fire(c + _NB - 1, (b + _NB - 1) % _NB)

            @pl.when(c < _CHUNKS)
            def _():
                drain(b)
                compute(c, b)

        return carry

    lax.fori_loop(0, (_CHUNKS + _NB - 1) // _NB, ring_body, 0)

    toff = pl.multiple_of(_CHUNKS * _E, 8)
    src_t = src_r.at[0, pl.ds(0, _TAIL), :]
    dst_t = dst_r.at[0, pl.ds(0, _TAIL), :]
    rel_t = rel_r.at[0, pl.ds(0, _TAIL), :]
    pltpu.async_copy(zn_hbm.at[src_i.at[pl.ds(toff, _TAIL)]], src_t, sem0)
    pltpu.async_copy(zn_hbm.at[dst_i.at[pl.ds(toff, _TAIL)]], dst_t, sem0)
    g = pltpu.async_copy(cos_hbm.at[typ_i.at[pl.ds(toff, _TAIL)]], rel_t, sem0)
    pltpu.make_async_copy(zn_hbm.at[pl.ds(0, _TAIL)], src_t, sem0).wait()
    pltpu.make_async_copy(zn_hbm.at[pl.ds(0, _TAIL)], dst_t, sem0).wait()
    g.wait()

    def t_tail(t, accs):
        a0, a1, col = accs
        sp = plsc.load_gather(src_r.at[0], [lane, col])
        up = plsc.load_gather(dst_r.at[0], [lane, col])
        rp = plsc.load_gather(rel_r.at[0], [lane, col])
        s0, s1 = plsc.unpack(plsc.bitcast(sp, jnp.bfloat16),
                             format=plsc.PackFormat.INTERLEAVED)
        u0, u1 = plsc.unpack(plsc.bitcast(up, jnp.bfloat16),
                             format=plsc.PackFormat.INTERLEAVED)
        r0, r1 = plsc.unpack(plsc.bitcast(rp, jnp.bfloat16),
                             format=plsc.PackFormat.INTERLEAVED)
        nxt = lax.bitwise_and(col + 1, _DP - 1)
        return (a0 + s0 * u0 * r0, a1 + s1 * u1 * r1, nxt)

    z16 = jnp.zeros((16,), jnp.float32)
    a0, a1, _ = lax.fori_loop(0, _DP, t_tail, (z16, z16, lane), unroll=4)
    out_v[pl.ds(_CHUNKS * _E, _TAIL)] = a0 + a1

    pltpu.sync_copy(out_v, out_hbm.at[pl.ds(base, _EPW)])


def kernel(z, phase_rel, edge_index, edge_type):
    zn, cosr = _precompute(z, phase_rel)
    zn_p = jax.lax.bitcast_convert_type(
        zn.astype(jnp.bfloat16).reshape(_N_NODES, _DP, 2), jnp.int32)
    cos_p = jax.lax.bitcast_convert_type(
        cosr.astype(jnp.bfloat16).reshape(_NUM_REL, _DP, 2), jnp.int32)
    src = edge_index[0]
    dst = edge_index[1]
    return _score_sc(zn_p, cos_p, src, dst, edge_type)


# speedup vs baseline: 1.0043x; 1.0043x over previous
import functools

import jax
import jax.numpy as jnp
from jax import lax
from jax.experimental import pallas as pl
from jax.experimental.pallas import tpu as pltpu
from jax.experimental.pallas import tpu_sc as plsc

_N_NODES = 10000
_N_EDGES = 320000
_D = 128
_NUM_REL = 1000

_NW = 32
_EPW = _N_EDGES // _NW
_E = 80
_NB = 4
_CHUNKS = _EPW // _E
_DP = _D // 2


def _precompute(z, phase_rel):
    def zn_body(z_ref, o_ref):
        x = z_ref[...]
        n = jnp.sqrt(jnp.sum(x * x, axis=1, keepdims=True))
        o_ref[...] = x / jnp.maximum(n, 1e-12)

    zn = pl.pallas_call(
        zn_body,
        out_shape=jax.ShapeDtypeStruct((_N_NODES, _D), jnp.float32),
        grid=(10,),
        in_specs=[pl.BlockSpec((_N_NODES // 10, _D), lambda i: (i, 0))],
        out_specs=pl.BlockSpec((_N_NODES // 10, _D), lambda i: (i, 0)),
    )(z)

    def cos_body(p_ref, o_ref):
        o_ref[...] = jnp.cos(p_ref[...])

    cosr = pl.pallas_call(
        cos_body,
        out_shape=jax.ShapeDtypeStruct((_NUM_REL, _D), jnp.float32),
    )(phase_rel)
    return zn, cosr


_mesh = plsc.VectorSubcoreMesh(core_axis_name="c", subcore_axis_name="s")


@functools.partial(
    pl.kernel,
    mesh=_mesh,
    compiler_params=pltpu.CompilerParams(needs_layout_passes=False,
                                         use_tc_tiling_on_sc=False),
    out_type=jax.ShapeDtypeStruct((_N_EDGES,), jnp.float32),
    scratch_types=[
        pltpu.VMEM((_EPW,), jnp.int32),
        pltpu.VMEM((_EPW,), jnp.int32),
        pltpu.VMEM((_EPW,), jnp.int32),
        pltpu.VMEM((_NB, _E, _DP), jnp.int32),
        pltpu.VMEM((_NB, _E, _DP), jnp.int32),
        pltpu.VMEM((_NB, _E, _DP), jnp.int32),
        pltpu.VMEM((_EPW,), jnp.float32),
        pltpu.SemaphoreType.DMA,
        pltpu.SemaphoreType.DMA,
        pltpu.SemaphoreType.DMA,
        pltpu.SemaphoreType.DMA,
    ],
)
def _score_sc(zn_hbm, cos_hbm, src_hbm, dst_hbm, typ_hbm, out_hbm,
              src_i, dst_i, typ_i, src_r, dst_r, rel_r, out_v,
              sem0, sem1, sem2, sem3):
    wid = lax.axis_index("s") * 2 + lax.axis_index("c")
    base = pl.multiple_of(wid * _EPW, _EPW)
    pltpu.sync_copy(src_hbm.at[pl.ds(base, _EPW)], src_i)
    pltpu.sync_copy(dst_hbm.at[pl.ds(base, _EPW)], dst_i)
    pltpu.sync_copy(typ_hbm.at[pl.ds(base, _EPW)], typ_i)
    sems = (sem0, sem1, sem2, sem3)
    lane = lax.iota(jnp.int32, 16)

    def fire(c, b):
        off = pl.multiple_of(c * _E, _E)
        pltpu.async_copy(zn_hbm.at[src_i.at[pl.ds(off, _E)]], src_r.at[b], sems[b])
        pltpu.async_copy(zn_hbm.at[dst_i.at[pl.ds(off, _E)]], dst_r.at[b], sems[b])
        pltpu.async_copy(cos_hbm.at[typ_i.at[pl.ds(off, _E)]], rel_r.at[b], sems[b])

    def drain(b):
        pltpu.make_async_copy(zn_hbm.at[pl.ds(0, _E)], src_r.at[b], sems[b]).wait()
        pltpu.make_async_copy(zn_hbm.at[pl.ds(0, _E)], dst_r.at[b], sems[b]).wait()
        pltpu.make_async_copy(cos_hbm.at[pl.ds(0, _E)], rel_r.at[b], sems[b]).wait()

    def compute(c, b):
        src_f = src_r.at[b]
        dst_f = dst_r.at[b]
        rel_f = rel_r.at[b]

        def unpk(x):
            return plsc.unpack(plsc.bitcast(x, jnp.bfloat16),
                               format=plsc.PackFormat.INTERLEAVED)

        def group_body(g, carry):
            rows = g * 16 + lane

            def t_body(t, accs):
                a0, a1, col = accs
                sp = plsc.load_gather(src_f, [rows, col])
                up = plsc.load_gather(dst_f, [rows, col])
                rp = plsc.load_gather(rel_f, [rows, col])
                s0, s1 = unpk(sp)
                u0, u1 = unpk(up)
                r0, r1 = unpk(rp)
                nxt = lax.bitwise_and(col + 1, _DP - 1)
                return (a0 + s0 * u0 * r0, a1 + s1 * u1 * r1, nxt)

            z16 = jnp.zeros((16,), jnp.float32)
            a0, a1, _ = lax.fori_loop(0, _DP, t_body, (z16, z16, lane),
                                      unroll=4)
            out_v[pl.ds(c * _E + g * 16, 16)] = a0 + a1
            return carry

        lax.fori_loop(0, _E // 16, group_body, 0)

    for b in range(_NB - 1):
        fire(b, b)

    def ring_body(t, carry):
        i = t * _NB
        for b in range(_NB):
            c = i + b

            @pl.when(c + _NB - 1 < _CHUNKS)
            def _():
                fire(c + _NB - 1, (b + _NB - 1) % _NB)

            @pl.when(c < _CHUNKS)
            def _():
                drain(b)
                compute(c, b)

        return carry

    lax.fori_loop(0, (_CHUNKS + _NB - 1) // _NB, ring_body, 0)
    pltpu.sync_copy(out_v, out_hbm.at[pl.ds(base, _EPW)])


def kernel(z, phase_rel, edge_index, edge_type):
    zn, cosr = _precompute(z, phase_rel)
    zn_p = jax.lax.bitcast_convert_type(
        zn.astype(jnp.bfloat16).reshape(_N_NODES, _DP, 2), jnp.int32)
    cos_p = jax.lax.bitcast_convert_type(
        cosr.astype(jnp.bfloat16).reshape(_NUM_REL, _DP, 2), jnp.int32)
    src = edge_index[0]
    dst = edge_index[1]
    return _score_sc(zn_p, cos_p, src, dst, edge_type)
